# Initial kernel scaffold; baseline (speedup 1.0000x reference)
import jax
import jax.numpy as jnp
from jax.experimental import pallas as pl


def kernel(input_ids, embed_table, W1, W2, g, W_head):
    raise NotImplementedError("write your pallas kernel here")


# speedup vs baseline: 1.1773x; 1.1773x over previous
import functools

import jax
import jax.numpy as jnp
from jax import lax
from jax.experimental import pallas as pl
from jax.experimental.pallas import tpu as pltpu
from jax.experimental.pallas import tpu_sc as plsc

S = 2048
D = 1024
FF = 2048
V = 32000
NL = 2
R = 256
TS = S // 2
TN = S - R

_HI = jax.lax.Precision.HIGHEST


def _dot(a, b, precision=None):
    return lax.dot_general(a, b, (((a.ndim - 1,), (0,)), ((), ())),
                           precision=precision,
                           preferred_element_type=jnp.float32)


def _rowT(vcol, ident):
    return lax.dot_general(vcol, ident, (((0,), (0,)), ((), ())),
                           precision=_HI, preferred_element_type=jnp.float32)


def _sc_gather(table, idx):
    n = idx.shape[0]
    d = table.shape[1]
    info = plsc.get_sparse_core_info()
    nc, ns = info.num_cores, info.num_subcores
    nw = nc * ns
    bpw = n // nw
    mesh = plsc.VectorSubcoreMesh(core_axis_name="c", subcore_axis_name="s")

    @functools.partial(
        pl.kernel, mesh=mesh,
        out_type=jax.ShapeDtypeStruct((n, d), jnp.float32),
        scratch_types=[
            pltpu.VMEM((bpw,), jnp.int32),
            pltpu.VMEM((bpw, d), jnp.float32),
            pltpu.SemaphoreType.DMA,
        ],
    )
    def k(idx_hbm, table_hbm, out_hbm, idx_v, rows_v, sem):
        wid = lax.axis_index("s") * nc + lax.axis_index("c")
        base = wid * bpw
        pltpu.sync_copy(idx_hbm.at[pl.ds(base, bpw)], idx_v)
        pltpu.async_copy(table_hbm.at[idx_v], rows_v, sem).wait()
        pltpu.sync_copy(rows_v, out_hbm.at[pl.ds(base, bpw)])

    return k(idx, table)


def _scores_body(dst_ref, src_ref, best_ref, bd_ref):
    dst = dst_ref[...]
    src = src_ref[...]
    dn = dst / (jnp.sqrt(jnp.sum(dst * dst, axis=-1, keepdims=True)) + 1e-12)
    sn = src / (jnp.sqrt(jnp.sum(src * src, axis=-1, keepdims=True)) + 1e-12)
    scores = lax.dot_general(sn, dn, (((1,), (1,)), ((), ())),
                             preferred_element_type=jnp.float32)
    best = jnp.max(scores, axis=-1, keepdims=True)
    it = lax.broadcasted_iota(jnp.int32, scores.shape, 1)
    bd = jnp.min(jnp.where(scores == best, it, jnp.int32(1 << 30)),
                 axis=-1, keepdims=True)
    best_ref[...] = best
    bd_ref[...] = bd


def _scores(dstv, srcv):
    blk = 128
    grid = TS // blk
    return pl.pallas_call(
        _scores_body,
        grid=(grid,),
        in_specs=[
            pl.BlockSpec((TS, D), lambda i: (0, 0)),
            pl.BlockSpec((blk, D), lambda i: (i, 0)),
        ],
        out_specs=[
            pl.BlockSpec((blk, 1), lambda i: (i, 0)),
            pl.BlockSpec((blk, 1), lambda i: (i, 0)),
        ],
        out_shape=[
            jax.ShapeDtypeStruct((TS, 1), jnp.float32),
            jax.ShapeDtypeStruct((TS, 1), jnp.int32),
        ],
    )(dstv, srcv)


def _idx1_body(best_ref, bd_ref, sd_ref, ss_ref, sel_ref,
               sdr_ref, ssr_ref, selr_ref, gbdr_ref):
    v = best_ref[...]
    bd = bd_ref[...].astype(jnp.float32)
    icol = lax.broadcasted_iota(jnp.int32, (TS, 1), 0)
    icol_f = icol.astype(jnp.float32)
    irow = lax.broadcasted_iota(jnp.int32, (TS, TS), 1)
    jcol = lax.broadcasted_iota(jnp.int32, (TS, TS), 0)
    ident = (irow == jcol).astype(jnp.float32)

    vrow = _rowT(v, ident)
    beats = (vrow > v) | ((vrow == v) & (irow < icol))
    rank = jnp.sum(beats.astype(jnp.float32), axis=-1, keepdims=True)
    sel = rank < float(R)
    sel_f = sel.astype(jnp.float32)
    notsel_f = 1.0 - sel_f

    lt = (irow < jcol).astype(jnp.float32)
    csk = _dot(lt, notsel_f, precision=_HI)
    sd = icol_f + csk
    ss = icol_f + 1.0 + csk

    bd_row = _rowT(bd, ident)
    sel_row = _rowT(sel_f, ident)
    rank_row = _rowT(rank, ident)
    m1 = (bd_row == jcol.astype(jnp.float32)) & (sel_row > 0.5)
    win_rank = jnp.max(jnp.where(m1, rank_row, -1.0), axis=-1, keepdims=True)
    matched = win_rank >= 0.0
    ws = jnp.min(jnp.where(m1 & (rank_row == win_rank),
                           irow, jnp.int32(1 << 30)),
                 axis=-1, keepdims=True).astype(jnp.float32)
    gb_d = jnp.where(matched, 2.0 * ws + 1.0, 2.0 * icol_f)

    sd_ref[...] = sd
    ss_ref[...] = ss
    sel_ref[...] = sel_f
    sdr_ref[...] = _rowT(sd, ident)
    ssr_ref[...] = _rowT(ss, ident)
    selr_ref[...] = sel_row
    gbdr_ref[...] = _rowT(gb_d, ident)


def _idx2_body(sdr_ref, ssr_ref, selr_ref, gbdr_ref, ga_ref, gb_ref):
    sd_row = sdr_ref[...]
    ss_row = ssr_ref[...]
    kept_src = selr_ref[...] < 0.5
    gbd_row = gbdr_ref[...]
    ipos = lax.broadcasted_iota(jnp.int32, (1, TS), 1).astype(jnp.float32)
    dpos = 2.0 * ipos
    spos = 2.0 * ipos + 1.0
    tcol = lax.broadcasted_iota(jnp.int32, (TN, 1), 0).astype(jnp.float32)
    od = sd_row == tcol
    os_ = (ss_row == tcol) & kept_src
    neg = jnp.float32(-1.0)
    ga_ref[...] = jnp.max(
        jnp.maximum(jnp.where(od, dpos, neg), jnp.where(os_, spos, neg)),
        axis=-1, keepdims=True)
    gb_ref[...] = jnp.max(
        jnp.maximum(jnp.where(od, gbd_row, neg), jnp.where(os_, spos, neg)),
        axis=-1, keepdims=True)


def _idx3_body(sd_ref, ss_ref, sel_ref, bd_ref, sdr_ref, gu_ref):
    sd = sd_ref[...]
    ss = ss_ref[...]
    sel = sel_ref[...] > 0.5
    bd = bd_ref[...].astype(jnp.float32)
    sd_row = sdr_ref[...]
    irow = lax.broadcasted_iota(jnp.int32, (TS, TS), 1)
    h1 = bd == irow.astype(jnp.float32)
    sdbd = jnp.max(jnp.where(h1, sd_row, jnp.float32(-1.0)),
                   axis=-1, keepdims=True)
    gu_odd = jnp.where(sel, sdbd, ss)
    gu_ref[...] = jnp.concatenate([sd, gu_odd], axis=1)


def _indices(best, bd):
    sd, ss, sel_f, sdr, ssr, selr, gbdr = pl.pallas_call(
        _idx1_body,
        out_shape=[jax.ShapeDtypeStruct((TS, 1), jnp.float32)] * 3
        + [jax.ShapeDtypeStruct((1, TS), jnp.float32)] * 4,
    )(best, bd)
    ga, gb = pl.pallas_call(
        _idx2_body,
        out_shape=[
            jax.ShapeDtypeStruct((TN, 1), jnp.float32),
            jax.ShapeDtypeStruct((TN, 1), jnp.float32),
        ],
    )(sdr, ssr, selr, gbdr)
    gu = pl.pallas_call(
        _idx3_body,
        out_shape=jax.ShapeDtypeStruct((TS, 2), jnp.float32),
    )(sd, ss, sel_f, bd, sdr)
    return ga, gb, gu


def _mlp_body(a_ref, b_ref, w1_ref, w2_ref, o_ref):
    xm = 0.5 * (a_ref[...] + b_ref[...])
    mid = jax.nn.gelu(_dot(xm, w1_ref[...]))
    o_ref[...] = xm + _dot(mid, w2_ref[...])


def _mlp(a, b, w1, w2):
    blk = 128
    grid = TN // blk
    return pl.pallas_call(
        _mlp_body,
        grid=(grid,),
        in_specs=[
            pl.BlockSpec((blk, D), lambda i: (i, 0)),
            pl.BlockSpec((blk, D), lambda i: (i, 0)),
            pl.BlockSpec((D, FF), lambda i: (0, 0)),
            pl.BlockSpec((FF, D), lambda i: (0, 0)),
        ],
        out_specs=pl.BlockSpec((blk, D), lambda i: (i, 0)),
        out_shape=jax.ShapeDtypeStruct((TN, D), jnp.float32),
    )(a, b, w1, w2)


def _rms_body(x_ref, g_ref, o_ref):
    x = x_ref[...]
    o_ref[...] = x * lax.rsqrt(jnp.mean(x * x, axis=-1, keepdims=True)
                               + 1e-6) * g_ref[...]


def _rms(x, g):
    return pl.pallas_call(
        _rms_body,
        out_shape=jax.ShapeDtypeStruct((S, D), jnp.float32),
    )(x, g.reshape(1, D))


_VBLK = 640
_VGRID = V // _VBLK


def _head_body(y_ref, w_ref, tgt_ref, logits_ref, loss_ref,
               m_ref, s_ref, tl_ref):
    i = pl.program_id(0)

    @pl.when(i == 0)
    def _():
        m_ref[...] = jnp.full((S, 1), -1e30, jnp.float32)
        s_ref[...] = jnp.zeros((S, 1), jnp.float32)
        tl_ref[...] = jnp.zeros((S, 1), jnp.float32)

    blk = _dot(y_ref[...], w_ref[...])
    logits_ref[...] = blk
    bmax = jnp.max(blk, axis=-1, keepdims=True)
    m_old = m_ref[...]
    m_new = jnp.maximum(m_old, bmax)
    e = jnp.exp(blk - m_new)
    s_ref[...] = s_ref[...] * jnp.exp(m_old - m_new) + jnp.sum(
        e, axis=-1, keepdims=True)
    m_ref[...] = m_new

    loc = tgt_ref[...] - i * _VBLK
    iv = lax.broadcasted_iota(jnp.int32, (S, _VBLK), 1)
    tl_ref[...] += jnp.sum(jnp.where(iv == loc, blk, 0.0),
                           axis=-1, keepdims=True)

    @pl.when(i == _VGRID - 1)
    def _():
        nll = m_ref[...] + jnp.log(s_ref[...]) - tl_ref[...]
        rmask = lax.broadcasted_iota(jnp.int32, (S, 1), 0) < (S - 1)
        loss_ref[...] = jnp.sum(jnp.where(rmask, nll, 0.0), axis=0,
                                keepdims=True) / float(S - 1)


def _head(y, w_head, tgt):
    return pl.pallas_call(
        _head_body,
        grid=(_VGRID,),
        in_specs=[
            pl.BlockSpec((S, D), lambda i: (0, 0)),
            pl.BlockSpec((D, _VBLK), lambda i: (0, i)),
            pl.BlockSpec((S, 1), lambda i: (0, 0)),
        ],
        out_specs=[
            pl.BlockSpec((S, _VBLK), lambda i: (0, i)),
            pl.BlockSpec((1, 1), lambda i: (0, 0)),
        ],
        out_shape=[
            jax.ShapeDtypeStruct((S, V), jnp.float32),
            jax.ShapeDtypeStruct((1, 1), jnp.float32),
        ],
        scratch_shapes=[
            pltpu.VMEM((S, 1), jnp.float32),
            pltpu.VMEM((S, 1), jnp.float32),
            pltpu.VMEM((S, 1), jnp.float32),
        ],
    )(y, w_head, tgt)


def kernel(input_ids, embed_table, W1, W2, g, W_head):
    ids = input_ids.reshape(S).astype(jnp.int32)
    x = _sc_gather(embed_table, ids)
    for l in range(NL):
        xr = x.reshape(TS, 2, D)
        dstv, srcv = xr[:, 0, :], xr[:, 1, :]
        best, bd = _scores(dstv, srcv)
        ga, gb, gu = _indices(best, bd)
        a = _sc_gather(x, ga.reshape(TN).astype(jnp.int32))
        b = _sc_gather(x, gb.reshape(TN).astype(jnp.int32))
        h = _mlp(a, b, W1[l], W2[l])
        x = _sc_gather(h, gu.reshape(S).astype(jnp.int32))
    y = _rms(x, g)
    tgt = jnp.concatenate([ids[1:], jnp.zeros((1,), jnp.int32)]).reshape(S, 1)
    logits, loss = _head(y, W_head, tgt)
    return (loss.reshape(()), logits.reshape(1, S, V))


# speedup vs baseline: 1.4408x; 1.2238x over previous
import functools

import jax
import jax.numpy as jnp
from jax import lax
from jax.experimental import pallas as pl
from jax.experimental.pallas import tpu as pltpu
from jax.experimental.pallas import tpu_sc as plsc

S = 2048
D = 1024
FF = 2048
V = 32000
NL = 2
R = 256
TS = S // 2
TN = S - R

_HI = jax.lax.Precision.HIGHEST


def _dot(a, b, precision=None):
    return lax.dot_general(a, b, (((a.ndim - 1,), (0,)), ((), ())),
                           precision=precision,
                           preferred_element_type=jnp.float32)


def _rowT(vcol, ident):
    return lax.dot_general(vcol, ident, (((0,), (0,)), ((), ())),
                           precision=_HI, preferred_element_type=jnp.float32)


def _sc_gather(table, idx):
    n = idx.shape[0]
    d = table.shape[1]
    info = plsc.get_sparse_core_info()
    nc, ns = info.num_cores, info.num_subcores
    nw = nc * ns
    bpw = n // nw
    mesh = plsc.VectorSubcoreMesh(core_axis_name="c", subcore_axis_name="s")

    @functools.partial(
        pl.kernel, mesh=mesh,
        out_type=jax.ShapeDtypeStruct((n, d), jnp.float32),
        scratch_types=[
            pltpu.VMEM((bpw,), jnp.int32),
            pltpu.VMEM((bpw, d), jnp.float32),
            pltpu.SemaphoreType.DMA,
        ],
    )
    def k(idx_hbm, table_hbm, out_hbm, idx_v, rows_v, sem):
        wid = lax.axis_index("s") * nc + lax.axis_index("c")
        base = wid * bpw
        pltpu.sync_copy(idx_hbm.at[pl.ds(base, bpw)], idx_v)
        pltpu.async_copy(table_hbm.at[idx_v], rows_v, sem).wait()
        pltpu.sync_copy(rows_v, out_hbm.at[pl.ds(base, bpw)])

    return k(idx, table)


def _scores_body(dst_ref, src_ref, best_ref, bd_ref):
    dst = dst_ref[...]
    src = src_ref[...]
    dn = dst / (jnp.sqrt(jnp.sum(dst * dst, axis=-1, keepdims=True)) + 1e-12)
    sn = src / (jnp.sqrt(jnp.sum(src * src, axis=-1, keepdims=True)) + 1e-12)
    scores = lax.dot_general(sn, dn, (((1,), (1,)), ((), ())),
                             preferred_element_type=jnp.float32)
    best = jnp.max(scores, axis=-1, keepdims=True)
    it = lax.broadcasted_iota(jnp.int32, scores.shape, 1)
    bd = jnp.min(jnp.where(scores == best, it, jnp.int32(1 << 30)),
                 axis=-1, keepdims=True)
    best_ref[...] = best
    bd_ref[...] = bd


def _scores(dstv, srcv):
    blk = 128
    grid = TS // blk
    return pl.pallas_call(
        _scores_body,
        grid=(grid,),
        in_specs=[
            pl.BlockSpec((TS, D), lambda i: (0, 0)),
            pl.BlockSpec((blk, D), lambda i: (i, 0)),
        ],
        out_specs=[
            pl.BlockSpec((blk, 1), lambda i: (i, 0)),
            pl.BlockSpec((blk, 1), lambda i: (i, 0)),
        ],
        out_shape=[
            jax.ShapeDtypeStruct((TS, 1), jnp.float32),
            jax.ShapeDtypeStruct((TS, 1), jnp.int32),
        ],
    )(dstv, srcv)


def _idx1_body(best_ref, bd_ref, sd_ref, ss_ref, sel_ref,
               sdr_ref, ssr_ref, selr_ref, gbdr_ref):
    v = best_ref[...]
    bd = bd_ref[...].astype(jnp.float32)
    icol = lax.broadcasted_iota(jnp.int32, (TS, 1), 0)
    icol_f = icol.astype(jnp.float32)
    irow = lax.broadcasted_iota(jnp.int32, (TS, TS), 1)
    jcol = lax.broadcasted_iota(jnp.int32, (TS, TS), 0)
    ident = (irow == jcol).astype(jnp.float32)

    vrow = _rowT(v, ident)
    beats = (vrow > v) | ((vrow == v) & (irow < icol))
    rank = jnp.sum(beats.astype(jnp.float32), axis=-1, keepdims=True)
    sel = rank < float(R)
    sel_f = sel.astype(jnp.float32)
    notsel_f = 1.0 - sel_f

    lt = (irow < jcol).astype(jnp.float32)
    csk = _dot(lt, notsel_f, precision=_HI)
    sd = icol_f + csk
    ss = icol_f + 1.0 + csk

    bd_row = _rowT(bd, ident)
    sel_row = _rowT(sel_f, ident)
    rank_row = _rowT(rank, ident)
    m1 = (bd_row == jcol.astype(jnp.float32)) & (sel_row > 0.5)
    win_rank = jnp.max(jnp.where(m1, rank_row, -1.0), axis=-1, keepdims=True)
    matched = win_rank >= 0.0
    ws = jnp.min(jnp.where(m1 & (rank_row == win_rank),
                           irow, jnp.int32(1 << 30)),
                 axis=-1, keepdims=True).astype(jnp.float32)
    gb_d = jnp.where(matched, 2.0 * ws + 1.0, 2.0 * icol_f)

    sd_ref[...] = sd
    ss_ref[...] = ss
    sel_ref[...] = sel_f
    sdr_ref[...] = _rowT(sd, ident)
    ssr_ref[...] = _rowT(ss, ident)
    selr_ref[...] = sel_row
    gbdr_ref[...] = _rowT(gb_d, ident)


def _idx2_body(sdr_ref, ssr_ref, selr_ref, gbdr_ref, ga_ref, gb_ref):
    sd_row = sdr_ref[...]
    ss_row = ssr_ref[...]
    kept_src = selr_ref[...] < 0.5
    gbd_row = gbdr_ref[...]
    ipos = lax.broadcasted_iota(jnp.int32, (1, TS), 1).astype(jnp.float32)
    dpos = 2.0 * ipos
    spos = 2.0 * ipos + 1.0
    tcol = lax.broadcasted_iota(jnp.int32, (TN, 1), 0).astype(jnp.float32)
    od = sd_row == tcol
    os_ = (ss_row == tcol) & kept_src
    neg = jnp.float32(-1.0)
    ga_ref[...] = jnp.max(
        jnp.maximum(jnp.where(od, dpos, neg), jnp.where(os_, spos, neg)),
        axis=-1, keepdims=True)
    gb_ref[...] = jnp.max(
        jnp.maximum(jnp.where(od, gbd_row, neg), jnp.where(os_, spos, neg)),
        axis=-1, keepdims=True)


def _idx3_body(sd_ref, ss_ref, sel_ref, bd_ref, sdr_ref, gu_ref):
    sd = sd_ref[...]
    ss = ss_ref[...]
    sel = sel_ref[...] > 0.5
    bd = bd_ref[...].astype(jnp.float32)
    sd_row = sdr_ref[...]
    irow = lax.broadcasted_iota(jnp.int32, (TS, TS), 1)
    h1 = bd == irow.astype(jnp.float32)
    sdbd = jnp.max(jnp.where(h1, sd_row, jnp.float32(-1.0)),
                   axis=-1, keepdims=True)
    gu_odd = jnp.where(sel, sdbd, ss)
    gu_ref[...] = jnp.concatenate([sd, gu_odd], axis=1)


def _indices(best, bd):
    sd, ss, sel_f, sdr, ssr, selr, gbdr = pl.pallas_call(
        _idx1_body,
        out_shape=[jax.ShapeDtypeStruct((TS, 1), jnp.float32)] * 3
        + [jax.ShapeDtypeStruct((1, TS), jnp.float32)] * 4,
    )(best, bd)
    ga, gb = pl.pallas_call(
        _idx2_body,
        out_shape=[
            jax.ShapeDtypeStruct((TN, 1), jnp.float32),
            jax.ShapeDtypeStruct((TN, 1), jnp.float32),
        ],
    )(sdr, ssr, selr, gbdr)
    gu = pl.pallas_call(
        _idx3_body,
        out_shape=jax.ShapeDtypeStruct((TS, 2), jnp.float32),
    )(sd, ss, sel_f, bd, sdr)
    return ga, gb, gu


def _mlp_body(a_ref, b_ref, w1_ref, w2_ref, o_ref):
    xm = 0.5 * (a_ref[...] + b_ref[...])
    mid = jax.nn.gelu(_dot(xm, w1_ref[...]))
    o_ref[...] = xm + _dot(mid, w2_ref[...])


def _mlp(a, b, w1, w2):
    blk = 256
    grid = TN // blk
    return pl.pallas_call(
        _mlp_body,
        grid=(grid,),
        in_specs=[
            pl.BlockSpec((blk, D), lambda i: (i, 0)),
            pl.BlockSpec((blk, D), lambda i: (i, 0)),
            pl.BlockSpec((D, FF), lambda i: (0, 0)),
            pl.BlockSpec((FF, D), lambda i: (0, 0)),
        ],
        out_specs=pl.BlockSpec((blk, D), lambda i: (i, 0)),
        out_shape=jax.ShapeDtypeStruct((TN, D), jnp.float32),
    )(a, b, w1, w2)


def _rms_body(x_ref, g_ref, o_ref):
    x = x_ref[...]
    o_ref[...] = x * lax.rsqrt(jnp.mean(x * x, axis=-1, keepdims=True)
                               + 1e-6) * g_ref[...]


def _rms(x, g):
    return pl.pallas_call(
        _rms_body,
        out_shape=jax.ShapeDtypeStruct((S, D), jnp.float32),
    )(x, g.reshape(1, D))


_VBLK = 1280
_VGRID = V // _VBLK


def _head_body(y_ref, w_ref, tgt_ref, logits_ref, loss_ref,
               m_ref, s_ref, tl_ref):
    i = pl.program_id(0)

    @pl.when(i == 0)
    def _():
        m_ref[...] = jnp.full((S, 1), -1e30, jnp.float32)
        s_ref[...] = jnp.zeros((S, 1), jnp.float32)
        tl_ref[...] = jnp.zeros((S, 1), jnp.float32)

    blk = _dot(y_ref[...], w_ref[...])
    logits_ref[...] = blk
    bmax = jnp.max(blk, axis=-1, keepdims=True)
    m_old = m_ref[...]
    m_new = jnp.maximum(m_old, bmax)
    e = jnp.exp(blk - m_new)
    s_ref[...] = s_ref[...] * jnp.exp(m_old - m_new) + jnp.sum(
        e, axis=-1, keepdims=True)
    m_ref[...] = m_new

    loc = tgt_ref[...] - i * _VBLK
    iv = lax.broadcasted_iota(jnp.int32, (S, _VBLK), 1)
    tl_ref[...] += jnp.sum(jnp.where(iv == loc, blk, 0.0),
                           axis=-1, keepdims=True)

    @pl.when(i == _VGRID - 1)
    def _():
        nll = m_ref[...] + jnp.log(s_ref[...]) - tl_ref[...]
        rmask = lax.broadcasted_iota(jnp.int32, (S, 1), 0) < (S - 1)
        loss_ref[...] = jnp.sum(jnp.where(rmask, nll, 0.0), axis=0,
                                keepdims=True) / float(S - 1)


def _head(y, w_head, tgt):
    return pl.pallas_call(
        _head_body,
        grid=(_VGRID,),
        in_specs=[
            pl.BlockSpec((S, D), lambda i: (0, 0)),
            pl.BlockSpec((D, _VBLK), lambda i: (0, i)),
            pl.BlockSpec((S, 1), lambda i: (0, 0)),
        ],
        out_specs=[
            pl.BlockSpec((S, _VBLK), lambda i: (0, i)),
            pl.BlockSpec((1, 1), lambda i: (0, 0)),
        ],
        out_shape=[
            jax.ShapeDtypeStruct((S, V), jnp.float32),
            jax.ShapeDtypeStruct((1, 1), jnp.float32),
        ],
        scratch_shapes=[
            pltpu.VMEM((S, 1), jnp.float32),
            pltpu.VMEM((S, 1), jnp.float32),
            pltpu.VMEM((S, 1), jnp.float32),
        ],
    )(y, w_head, tgt)


def kernel(input_ids, embed_table, W1, W2, g, W_head):
    ids = input_ids.reshape(S).astype(jnp.int32)
    x = _sc_gather(embed_table, ids)
    for l in range(NL):
        xr = x.reshape(TS, 2, D)
        dstv, srcv = xr[:, 0, :], xr[:, 1, :]
        best, bd = _scores(dstv, srcv)
        ga, gb, gu = _indices(best, bd)
        a = _sc_gather(x, ga.reshape(TN).astype(jnp.int32))
        b = _sc_gather(x, gb.reshape(TN).astype(jnp.int32))
        h = _mlp(a, b, W1[l], W2[l])
        x = _sc_gather(h, gu.reshape(S).astype(jnp.int32))
    y = _rms(x, g)
    tgt = jnp.concatenate([ids[1:], jnp.zeros((1,), jnp.int32)]).reshape(S, 1)
    logits, loss = _head(y, W_head, tgt)
    return (loss.reshape(()), logits.reshape(1, S, V))
